# baseline (speedup 1.0000x reference)
import functools

import jax
import jax.numpy as jnp
from jax import lax
from jax.experimental import pallas as pl
from jax.experimental.pallas import tpu as pltpu
from jax.experimental.pallas import tpu_sc as plsc

_N = 1_000_000
_D = 64
_Q = 64
_C = 32_768
_GRID = -(-_N // _C)
_LAST_VALID = _N - (_GRID - 1) * _C
_EPS = 1e-12


def _topk_body(q_ref, mt_ref, conf_ref, idx_ref, qn_buf):
    i = pl.program_id(0)

    @pl.when(i == 0)
    def _init():
        conf_ref[...] = jnp.full((1, _Q), -jnp.inf, jnp.float32)
        idx_ref[...] = jnp.zeros((1, _Q), jnp.int32)
        q = q_ref[...]
        qn_buf[...] = q / jnp.maximum(
            jnp.sqrt(jnp.sum(q * q, axis=1, keepdims=True)), _EPS)

    qn = qn_buf[...]
    mt = mt_ref[...]
    colf = lax.broadcasted_iota(jnp.int32, (1, _C), 1).astype(jnp.float32)

    n = jnp.maximum(jnp.sqrt(jnp.sum(mt * mt, axis=0, keepdims=True)), _EPS)
    thresh = jnp.where(i == _GRID - 1, float(_LAST_VALID), jnp.inf)
    n = jnp.where(colf >= thresh, jnp.inf, n)
    mn = mt * (1.0 / n)
    sims = lax.dot_general(qn, mn, (((1,), (0,)), ((), ())),
                           preferred_element_type=jnp.float32)

    local_max = jnp.max(sims, axis=1)
    local_arg = jnp.min(
        jnp.where(sims == local_max[:, None], colf, float(_C)),
        axis=1).astype(jnp.int32)

    run_v = conf_ref[0, :]
    upd = local_max > run_v
    conf_ref[0, :] = jnp.where(upd, local_max, run_v)
    idx_ref[0, :] = jnp.where(upd, i * _C + local_arg, idx_ref[0, :])


_topk_call = pl.pallas_call(
    _topk_body,
    grid=(_GRID,),
    in_specs=[
        pl.BlockSpec((_Q, _D), lambda i: (0, 0)),
        pl.BlockSpec((_D, _C), lambda i: (0, i)),
    ],
    out_specs=[
        pl.BlockSpec((1, _Q), lambda i: (0, 0)),
        pl.BlockSpec((1, _Q), lambda i: (0, 0)),
    ],
    out_shape=[
        jax.ShapeDtypeStruct((1, _Q), jnp.float32),
        jax.ShapeDtypeStruct((1, _Q), jnp.int32),
    ],
    scratch_shapes=[pltpu.VMEM((_Q, _D), jnp.float32)],
)


def _sc_gather_body(values_hbm, idx_hbm, out_hbm, idx_v, rows_v, sem):
    wid = lax.axis_index("s") * 2 + lax.axis_index("c")

    @pl.when(wid == 0)
    def _():
        pltpu.sync_copy(idx_hbm, idx_v)
        pltpu.async_copy(values_hbm.at[idx_v], rows_v, sem).wait()
        pltpu.sync_copy(rows_v, out_hbm)


_sc_gather = functools.partial(
    pl.kernel,
    out_type=jax.ShapeDtypeStruct((_Q,), jnp.int32),
    mesh=plsc.VectorSubcoreMesh(core_axis_name="c", subcore_axis_name="s"),
    scratch_types=[
        pltpu.VMEM((_Q,), jnp.int32),
        pltpu.VMEM((_Q,), jnp.int32),
        pltpu.SemaphoreType.DMA,
    ],
)(_sc_gather_body)


def kernel(query_features, memory_keys, memory_values):
    mt = memory_keys.T
    conf2, idx2 = _topk_call(query_features, mt)
    confidence = conf2[0]
    indices = idx2[0]
    retrieved = _sc_gather(memory_values, indices)
    return retrieved, confidence
